# baseline (speedup 1.0000x reference)
import jax
import jax.numpy as jnp
from jax import lax
from jax.experimental import pallas as pl
from jax.experimental.pallas import tpu as pltpu
from jax.experimental.pallas import tpu_sc as plsc

_NC, _NS = 2, 16
_NW = _NC * _NS
_SZ = [16384, 4096, 1024, 256, 64]
_SEQ = 9
_C = [3, 32, 32, 32, 64]
_BS = 16


def _mesh():
    return plsc.VectorSubcoreMesh(core_axis_name="c", subcore_axis_name="s",
                                  num_cores=_NC, num_subcores=_NS)


def _chunk(b_per_w, D, budget):
    for c in range(min(128, b_per_w), 0, -8):
        if b_per_w % c == 0 and c * D * 4 <= budget:
            return c
    raise AssertionError((b_per_w, D))


def _sc_gather(D, B):
    b_per_w = B // _NW
    assert B % _NW == 0 and b_per_w % 8 == 0, (B,)
    ch = _chunk(b_per_w, D, 393216)
    n_chunks = b_per_w // ch

    def body(table_hbm, idx_hbm, out_hbm, idx_v, rows_v, sem):
        wid = lax.axis_index("s") * _NC + lax.axis_index("c")
        w0 = wid * b_per_w

        def step(ci, carry):
            base = w0 + ci * ch
            pltpu.sync_copy(idx_hbm.at[pl.ds(base, ch)], idx_v)
            pltpu.async_copy(table_hbm.at[idx_v], rows_v, sem).wait()
            pltpu.sync_copy(rows_v, out_hbm.at[pl.ds(base, ch)])
            return carry

        if n_chunks == 1:
            step(0, 0)
        else:
            lax.fori_loop(0, n_chunks, step, 0)

    return pl.kernel(body,
                     out_type=jax.ShapeDtypeStruct((B, D), jnp.float32),
                     mesh=_mesh(),
                     scratch_types=[pltpu.VMEM((ch,), jnp.int32),
                                    pltpu.VMEM((ch, D), jnp.float32),
                                    pltpu.SemaphoreType.DMA])


def _sc_sum9(D, N, elu):
    n_per_w = N // _NW
    assert N % _NW == 0 and n_per_w % 8 == 0, (N,)
    ch = _chunk(n_per_w, D, 360000 // (_SEQ + 1))
    n_chunks = n_per_w // ch

    def body(z_hbm, idx_hbm, b_hbm, out_hbm, idx_v, buf9, obuf, bias_v, sem):
        wid = lax.axis_index("s") * _NC + lax.axis_index("c")
        w0 = wid * n_per_w
        pltpu.sync_copy(b_hbm, bias_v)

        def step(ci, carry):
            base = w0 + ci * ch
            for k in range(_SEQ):
                pltpu.sync_copy(idx_hbm.at[pl.ds(k * N + base, ch)],
                                idx_v.at[k])
            cps = [pltpu.async_copy(z_hbm.at[idx_v.at[k]], buf9.at[k], sem)
                   for k in range(_SEQ)]
            for cp in cps:
                cp.wait()

            def rowf(r, cc):
                for d in range(D // 16):
                    sl = pl.ds(d * 16, 16)
                    xv = buf9[0, r, sl] + bias_v[sl]
                    for k in range(1, _SEQ):
                        xv = xv + buf9[k, r, sl]
                    if elu:
                        xv = jnp.where(xv > 0.0, xv,
                                       jnp.exp(jnp.minimum(xv, 0.0)) - 1.0)
                    obuf[r, sl] = xv
                return cc

            lax.fori_loop(0, ch, rowf, 0)
            pltpu.sync_copy(obuf, out_hbm.at[pl.ds(base, ch)])
            return carry

        if n_chunks == 1:
            step(0, 0)
        else:
            lax.fori_loop(0, n_chunks, step, 0)

    return pl.kernel(body,
                     out_type=jax.ShapeDtypeStruct((N, D), jnp.float32),
                     mesh=_mesh(),
                     scratch_types=[pltpu.VMEM((_SEQ, ch), jnp.int32),
                                    pltpu.VMEM((_SEQ, ch, D), jnp.float32),
                                    pltpu.VMEM((ch, D), jnp.float32),
                                    pltpu.VMEM((D,), jnp.float32),
                                    pltpu.SemaphoreType.DMA])


def _z_tc(X, Ws):
    R, Cin = X.shape
    S, _, Cout = Ws.shape
    T = min(2048, R)

    def body(x_ref, w_ref, o_ref):
        o_ref[0] = jnp.dot(x_ref[...], w_ref[0],
                           preferred_element_type=jnp.float32)

    return pl.pallas_call(
        body,
        grid=(R // T, S),
        in_specs=[pl.BlockSpec((T, Cin), lambda i, s: (i, 0)),
                  pl.BlockSpec((1, Cin, Cout), lambda i, s: (s, 0, 0))],
        out_specs=pl.BlockSpec((1, T, Cout), lambda i, s: (s, i, 0)),
        out_shape=jax.ShapeDtypeStruct((S, R, Cout), jnp.float32),
    )(X, Ws)


def _conv_tc(G, Ws, brow, elu):
    S, R, Cin = G.shape
    Cout = Ws.shape[2]
    T = min(2048, R)

    def body(g_ref, w_ref, b_ref, o_ref):
        acc = jnp.dot(g_ref[0], w_ref[0], preferred_element_type=jnp.float32)
        for s in range(1, S):
            acc = acc + jnp.dot(g_ref[s], w_ref[s],
                                preferred_element_type=jnp.float32)
        acc = acc + b_ref[...]
        if elu:
            acc = jnp.where(acc > 0, acc, jnp.exp(jnp.minimum(acc, 0.0)) - 1.0)
        o_ref[...] = acc

    return pl.pallas_call(
        body,
        grid=(R // T,),
        in_specs=[pl.BlockSpec((S, T, Cin), lambda i: (0, i, 0)),
                  pl.BlockSpec((S, Cin, Cout), lambda i: (0, 0, 0)),
                  pl.BlockSpec((1, Cout), lambda i: (0, 0))],
        out_specs=pl.BlockSpec((T, Cout), lambda i: (i, 0)),
        out_shape=jax.ShapeDtypeStruct((R, Cout), jnp.float32),
    )(G, Ws, brow)


def _pool_tc(G4, val4):
    _, N, D = G4.shape
    T = min(512 if D <= 512 else 256, N)

    def body(g_ref, v_ref, o_ref):
        acc = g_ref[0] * v_ref[0]
        for k in range(1, 4):
            acc = acc + g_ref[k] * v_ref[k]
        o_ref[...] = acc

    return pl.pallas_call(
        body,
        grid=(N // T,),
        in_specs=[pl.BlockSpec((4, T, D), lambda i: (0, i, 0)),
                  pl.BlockSpec((4, T, 1), lambda i: (0, i, 0))],
        out_specs=pl.BlockSpec((T, D), lambda i: (i, 0)),
        out_shape=jax.ShapeDtypeStruct((N, D), jnp.float32),
    )(G4, val4)


def _fc_tc(h, w1t, b1, w2t, b2):
    def body(h_ref, w1_ref, b1_ref, w2_ref, b2_ref, o_ref):
        mu = jnp.dot(h_ref[...], w1_ref[...],
                     preferred_element_type=jnp.float32) + b1_ref[...]
        mu = jax.nn.sigmoid(mu)
        o_ref[...] = jnp.dot(mu, w2_ref[...],
                             preferred_element_type=jnp.float32) + b2_ref[...]

    return pl.pallas_call(
        body,
        out_shape=jax.ShapeDtypeStruct((h.shape[0], w2t.shape[1]),
                                       jnp.float32),
    )(h, w1t, b1, w2t, b2)


def _wstack(W, Cin, Cout):
    return W.reshape(Cout, _SEQ, Cin).transpose(1, 2, 0)


def _idx9(sp, N):
    return (jnp.arange(_SEQ, dtype=jnp.int32)[:, None] * N
            + jnp.transpose(sp)).reshape(-1)


def kernel(x, sp_idx_0, sp_idx_1, sp_idx_2, sp_idx_3,
           dn_row_0, dn_row_1, dn_row_2, dn_row_3,
           dn_col_0, dn_col_1, dn_col_2, dn_col_3,
           dn_val_0, dn_val_1, dn_val_2, dn_val_3,
           up_row_0, up_row_1, up_row_2, up_row_3,
           up_col_0, up_col_1, up_col_2, up_col_3,
           up_val_0, up_val_1, up_val_2, up_val_3,
           enW0, enW1, enW2, enW3, enb0, enb1, enb2, enb3,
           en_fcW, en_fcb, de_fcW, de_fcb,
           deW0, deW1, deW2, deW3, deb0, deb1, deb2, deb3,
           outW, outb):
    sp = [sp_idx_0, sp_idx_1, sp_idx_2, sp_idx_3]
    enW = [enW0, enW1, enW2, enW3]
    enb = [enb0, enb1, enb2, enb3]
    deW = [deW0, deW1, deW2, deW3]
    deb = [deb0, deb1, deb2, deb3]
    dncol = [dn_col_0, dn_col_1, dn_col_2, dn_col_3]
    dnval = [dn_val_0, dn_val_1, dn_val_2, dn_val_3]
    upcol = [up_col_0, up_col_1, up_col_2, up_col_3]
    upval = [up_val_0, up_val_1, up_val_2, up_val_3]

    x8 = jnp.pad(jnp.transpose(x, (1, 0, 2)), ((0, 0), (0, 0), (0, 5)))
    h = x8.reshape(_SZ[0], _BS * 8)

    for i in range(4):
        N, Cin, Cout = _SZ[i], _C[i], _C[i + 1]
        if i == 0:
            G = _sc_gather(_BS * 8, _SEQ * N)(h, jnp.transpose(sp[0])
                                              .reshape(-1))
            Ws = jnp.pad(_wstack(enW[0], Cin, Cout), ((0, 0), (0, 5), (0, 0)))
            conv = _conv_tc(G.reshape(_SEQ, N * _BS, 8), Ws,
                            enb[0].reshape(1, Cout), elu=True)
            conv = conv.reshape(N, _BS * Cout)
        else:
            D = _BS * Cout
            Z = _z_tc(h.reshape(N * _BS, Cin), _wstack(enW[i], Cin, Cout))
            conv = _sc_sum9(D, N, elu=True)(
                Z.reshape(_SEQ * N, D), _idx9(sp[i], N),
                jnp.tile(enb[i], _BS))
        M = _SZ[i + 1]
        D = _BS * Cout
        colT = jnp.transpose(dncol[i].reshape(M, 4)).reshape(-1)
        val4 = jnp.transpose(dnval[i].reshape(M, 4)).reshape(4, M, 1)
        G4 = _sc_gather(D, 4 * M)(conv, colT)
        h = _pool_tc(G4.reshape(4, M, D), val4)

    hflat = jnp.transpose(h.reshape(_SZ[4], _BS, _C[4]),
                          (1, 0, 2)).reshape(_BS, _SZ[4] * _C[4])
    z = _fc_tc(hflat, en_fcW.T, en_fcb.reshape(1, -1),
               de_fcW.T, de_fcb.reshape(1, -1))
    h = jnp.transpose(z.reshape(_BS, _SZ[4], _C[4]),
                      (1, 0, 2)).reshape(_SZ[4], _BS * _C[4])

    dec_cin = [64, 64, 32, 32]
    dec_cout = [64, 32, 32, 32]
    for j in range(4):
        lvl = 3 - j
        N, M = _SZ[lvl], _SZ[lvl + 1]
        Cin, Cout = dec_cin[j], dec_cout[j]
        D = _BS * Cin
        colT = jnp.transpose(upcol[lvl].reshape(N, 4)).reshape(-1)
        val4 = jnp.transpose(upval[lvl].reshape(N, 4)).reshape(4, N, 1)
        G4 = _sc_gather(D, 4 * N)(h, colT)
        hp = _pool_tc(G4.reshape(4, N, D), val4)
        Do = _BS * Cout
        Z = _z_tc(hp.reshape(N * _BS, Cin), _wstack(deW[j], Cin, Cout))
        h = _sc_sum9(Do, N, elu=True)(
            Z.reshape(_SEQ * N, Do), _idx9(sp[lvl], N),
            jnp.tile(deb[j], _BS))

    N = _SZ[0]
    Wo = jnp.pad(_wstack(outW, 32, 3), ((0, 0), (0, 0), (0, 5)))
    Z = _z_tc(h.reshape(N * _BS, 32), Wo)
    bias8 = jnp.tile(jnp.pad(outb, (0, 5)), _BS)
    out = _sc_sum9(_BS * 8, N, elu=False)(
        Z.reshape(_SEQ * N, _BS * 8), _idx9(sp[0], N), bias8)
    out = out.reshape(N, _BS, 8)[:, :, :3]
    return jnp.transpose(out, (1, 0, 2))


# speedup vs baseline: 1.2892x; 1.2892x over previous
import jax
import jax.numpy as jnp
from jax import lax
from jax.experimental import pallas as pl
from jax.experimental.pallas import tpu as pltpu
from jax.experimental.pallas import tpu_sc as plsc

_NC, _NS = 2, 16
_NW = _NC * _NS
_SZ = [16384, 4096, 1024, 256, 64]
_SEQ = 9
_C = [3, 32, 32, 32, 64]
_BS = 16


def _mesh():
    return plsc.VectorSubcoreMesh(core_axis_name="c", subcore_axis_name="s",
                                  num_cores=_NC, num_subcores=_NS)


def _chunk(b_per_w, D, budget):
    for c in range(min(128, b_per_w), 0, -8):
        if b_per_w % c == 0 and c * D * 4 <= budget:
            return c
    raise AssertionError((b_per_w, D))


def _sc_gather(D, B):
    b_per_w = B // _NW
    assert B % _NW == 0 and b_per_w % 8 == 0, (B,)
    ch = _chunk(b_per_w, D, 393216)
    n_chunks = b_per_w // ch

    def body(table_hbm, idx_hbm, out_hbm, idx_v, rows_v, sem):
        wid = lax.axis_index("s") * _NC + lax.axis_index("c")
        w0 = wid * b_per_w

        def step(ci, carry):
            base = w0 + ci * ch
            pltpu.sync_copy(idx_hbm.at[pl.ds(base, ch)], idx_v)
            pltpu.async_copy(table_hbm.at[idx_v], rows_v, sem).wait()
            pltpu.sync_copy(rows_v, out_hbm.at[pl.ds(base, ch)])
            return carry

        if n_chunks == 1:
            step(0, 0)
        else:
            lax.fori_loop(0, n_chunks, step, 0)

    return pl.kernel(body,
                     out_type=jax.ShapeDtypeStruct((B, D), jnp.float32),
                     mesh=_mesh(),
                     scratch_types=[pltpu.VMEM((ch,), jnp.int32),
                                    pltpu.VMEM((ch, D), jnp.float32),
                                    pltpu.SemaphoreType.DMA])


def _sum9_tc(G, brow, elu):
    S, N, D = G.shape
    T = min(2048 if D <= 128 else 512, N)

    def body(g_ref, b_ref, o_ref):
        acc = g_ref[0] + b_ref[...]
        for s in range(1, S):
            acc = acc + g_ref[s]
        if elu:
            acc = jnp.where(acc > 0, acc, jnp.exp(jnp.minimum(acc, 0.0)) - 1.0)
        o_ref[...] = acc

    return pl.pallas_call(
        body,
        grid=(N // T,),
        in_specs=[pl.BlockSpec((S, T, D), lambda i: (0, i, 0)),
                  pl.BlockSpec((1, D), lambda i: (0, 0))],
        out_specs=pl.BlockSpec((T, D), lambda i: (i, 0)),
        out_shape=jax.ShapeDtypeStruct((N, D), jnp.float32),
    )(G, brow)


def _z_tc(X, Ws):
    R, Cin = X.shape
    S, _, Cout = Ws.shape
    T = min(2048, R)

    def body(x_ref, w_ref, o_ref):
        o_ref[0] = jnp.dot(x_ref[...], w_ref[0],
                           preferred_element_type=jnp.float32)

    return pl.pallas_call(
        body,
        grid=(R // T, S),
        in_specs=[pl.BlockSpec((T, Cin), lambda i, s: (i, 0)),
                  pl.BlockSpec((1, Cin, Cout), lambda i, s: (s, 0, 0))],
        out_specs=pl.BlockSpec((1, T, Cout), lambda i, s: (s, i, 0)),
        out_shape=jax.ShapeDtypeStruct((S, R, Cout), jnp.float32),
    )(X, Ws)


def _conv_tc(G, Ws, brow, elu):
    S, R, Cin = G.shape
    Cout = Ws.shape[2]
    T = min(2048, R)

    def body(g_ref, w_ref, b_ref, o_ref):
        acc = jnp.dot(g_ref[0], w_ref[0], preferred_element_type=jnp.float32)
        for s in range(1, S):
            acc = acc + jnp.dot(g_ref[s], w_ref[s],
                                preferred_element_type=jnp.float32)
        acc = acc + b_ref[...]
        if elu:
            acc = jnp.where(acc > 0, acc, jnp.exp(jnp.minimum(acc, 0.0)) - 1.0)
        o_ref[...] = acc

    return pl.pallas_call(
        body,
        grid=(R // T,),
        in_specs=[pl.BlockSpec((S, T, Cin), lambda i: (0, i, 0)),
                  pl.BlockSpec((S, Cin, Cout), lambda i: (0, 0, 0)),
                  pl.BlockSpec((1, Cout), lambda i: (0, 0))],
        out_specs=pl.BlockSpec((T, Cout), lambda i: (i, 0)),
        out_shape=jax.ShapeDtypeStruct((R, Cout), jnp.float32),
    )(G, Ws, brow)


def _pool_tc(G4, val4):
    _, N, D = G4.shape
    T = min(512 if D <= 512 else 256, N)

    def body(g_ref, v_ref, o_ref):
        acc = g_ref[0] * v_ref[0]
        for k in range(1, 4):
            acc = acc + g_ref[k] * v_ref[k]
        o_ref[...] = acc

    return pl.pallas_call(
        body,
        grid=(N // T,),
        in_specs=[pl.BlockSpec((4, T, D), lambda i: (0, i, 0)),
                  pl.BlockSpec((4, T, 1), lambda i: (0, i, 0))],
        out_specs=pl.BlockSpec((T, D), lambda i: (i, 0)),
        out_shape=jax.ShapeDtypeStruct((N, D), jnp.float32),
    )(G4, val4)


def _fc_tc(h, w1t, b1, w2t, b2):
    def body(h_ref, w1_ref, b1_ref, w2_ref, b2_ref, o_ref):
        mu = jnp.dot(h_ref[...], w1_ref[...],
                     preferred_element_type=jnp.float32) + b1_ref[...]
        mu = jax.nn.sigmoid(mu)
        o_ref[...] = jnp.dot(mu, w2_ref[...],
                             preferred_element_type=jnp.float32) + b2_ref[...]

    return pl.pallas_call(
        body,
        out_shape=jax.ShapeDtypeStruct((h.shape[0], w2t.shape[1]),
                                       jnp.float32),
    )(h, w1t, b1, w2t, b2)


def _wstack(W, Cin, Cout):
    return W.reshape(Cout, _SEQ, Cin).transpose(1, 2, 0)


def _idx9(sp, N):
    return (jnp.arange(_SEQ, dtype=jnp.int32)[:, None] * N
            + jnp.transpose(sp)).reshape(-1)


def kernel(x, sp_idx_0, sp_idx_1, sp_idx_2, sp_idx_3,
           dn_row_0, dn_row_1, dn_row_2, dn_row_3,
           dn_col_0, dn_col_1, dn_col_2, dn_col_3,
           dn_val_0, dn_val_1, dn_val_2, dn_val_3,
           up_row_0, up_row_1, up_row_2, up_row_3,
           up_col_0, up_col_1, up_col_2, up_col_3,
           up_val_0, up_val_1, up_val_2, up_val_3,
           enW0, enW1, enW2, enW3, enb0, enb1, enb2, enb3,
           en_fcW, en_fcb, de_fcW, de_fcb,
           deW0, deW1, deW2, deW3, deb0, deb1, deb2, deb3,
           outW, outb):
    sp = [sp_idx_0, sp_idx_1, sp_idx_2, sp_idx_3]
    enW = [enW0, enW1, enW2, enW3]
    enb = [enb0, enb1, enb2, enb3]
    deW = [deW0, deW1, deW2, deW3]
    deb = [deb0, deb1, deb2, deb3]
    dncol = [dn_col_0, dn_col_1, dn_col_2, dn_col_3]
    dnval = [dn_val_0, dn_val_1, dn_val_2, dn_val_3]
    upcol = [up_col_0, up_col_1, up_col_2, up_col_3]
    upval = [up_val_0, up_val_1, up_val_2, up_val_3]

    x8 = jnp.pad(jnp.transpose(x, (1, 0, 2)), ((0, 0), (0, 0), (0, 5)))
    h = x8.reshape(_SZ[0], _BS * 8)

    for i in range(4):
        N, Cin, Cout = _SZ[i], _C[i], _C[i + 1]
        if i == 0:
            G = _sc_gather(_BS * 8, _SEQ * N)(h, jnp.transpose(sp[0])
                                              .reshape(-1))
            Ws = jnp.pad(_wstack(enW[0], Cin, Cout), ((0, 0), (0, 5), (0, 0)))
            conv = _conv_tc(G.reshape(_SEQ, N * _BS, 8), Ws,
                            enb[0].reshape(1, Cout), elu=True)
            conv = conv.reshape(N, _BS * Cout)
        else:
            D = _BS * Cin
            G = _sc_gather(D, _SEQ * N)(h, jnp.transpose(sp[i]).reshape(-1))
            conv = _conv_tc(G.reshape(_SEQ, N * _BS, Cin),
                            _wstack(enW[i], Cin, Cout),
                            enb[i].reshape(1, Cout), elu=True)
            conv = conv.reshape(N, _BS * Cout)
        M = _SZ[i + 1]
        D = _BS * Cout
        colT = jnp.transpose(dncol[i].reshape(M, 4)).reshape(-1)
        val4 = jnp.transpose(dnval[i].reshape(M, 4)).reshape(4, M, 1)
        G4 = _sc_gather(D, 4 * M)(conv, colT)
        h = _pool_tc(G4.reshape(4, M, D), val4)

    hflat = jnp.transpose(h.reshape(_SZ[4], _BS, _C[4]),
                          (1, 0, 2)).reshape(_BS, _SZ[4] * _C[4])
    z = _fc_tc(hflat, en_fcW.T, en_fcb.reshape(1, -1),
               de_fcW.T, de_fcb.reshape(1, -1))
    h = jnp.transpose(z.reshape(_BS, _SZ[4], _C[4]),
                      (1, 0, 2)).reshape(_SZ[4], _BS * _C[4])

    dec_cin = [64, 64, 32, 32]
    dec_cout = [64, 32, 32, 32]
    for j in range(4):
        lvl = 3 - j
        N, M = _SZ[lvl], _SZ[lvl + 1]
        Cin, Cout = dec_cin[j], dec_cout[j]
        D = _BS * Cin
        colT = jnp.transpose(upcol[lvl].reshape(N, 4)).reshape(-1)
        val4 = jnp.transpose(upval[lvl].reshape(N, 4)).reshape(4, N, 1)
        G4 = _sc_gather(D, 4 * N)(h, colT)
        hp = _pool_tc(G4.reshape(4, N, D), val4)
        Do = _BS * Cout
        if Cout < Cin:
            Z = _z_tc(hp.reshape(N * _BS, Cin), _wstack(deW[j], Cin, Cout))
            G = _sc_gather(Do, _SEQ * N)(Z.reshape(_SEQ * N, Do),
                                         _idx9(sp[lvl], N))
            h = _sum9_tc(G.reshape(_SEQ, N, Do),
                         jnp.tile(deb[j], _BS).reshape(1, Do), elu=True)
        else:
            G = _sc_gather(D, _SEQ * N)(hp, jnp.transpose(sp[lvl])
                                        .reshape(-1))
            conv = _conv_tc(G.reshape(_SEQ, N * _BS, Cin),
                            _wstack(deW[j], Cin, Cout),
                            deb[j].reshape(1, Cout), elu=True)
            h = conv.reshape(N, _BS * Cout)

    N = _SZ[0]
    Wo = jnp.pad(_wstack(outW, 32, 3), ((0, 0), (0, 0), (0, 5)))
    Z = _z_tc(h.reshape(N * _BS, 32), Wo)
    bias8 = jnp.tile(jnp.pad(outb, (0, 5)), _BS).reshape(1, _BS * 8)
    G = _sc_gather(_BS * 8, _SEQ * N)(Z.reshape(_SEQ * N, _BS * 8),
                                      _idx9(sp[0], N))
    out = _sum9_tc(G.reshape(_SEQ, N, _BS * 8), bias8, elu=False)
    out = out.reshape(N, _BS, 8)[:, :, :3]
    return jnp.transpose(out, (1, 0, 2))


# speedup vs baseline: 4.0016x; 3.1039x over previous
import jax
import jax.numpy as jnp
from jax import lax
from jax.experimental import pallas as pl
from jax.experimental.pallas import tpu as pltpu
from jax.experimental.pallas import tpu_sc as plsc

_NC, _NS = 2, 16
_NW = _NC * _NS
_SZ = [16384, 4096, 1024, 256, 64]
_SEQ = 9
_C = [3, 32, 32, 32, 64]
_BS = 16


def _mesh():
    return plsc.VectorSubcoreMesh(core_axis_name="c", subcore_axis_name="s",
                                  num_cores=_NC, num_subcores=_NS)


def _chunk(b_per_w, D, budget):
    for c in range(min(128, b_per_w), 0, -8):
        if b_per_w % c == 0 and c * D * 4 <= budget:
            return c
    raise AssertionError((b_per_w, D))


def _sc_gather(D, B):
    b_per_w = B // _NW
    assert B % _NW == 0 and b_per_w % 8 == 0, (B,)
    ch = _chunk(b_per_w, D, 393216)
    n_chunks = b_per_w // ch

    def body(table_hbm, idx_hbm, out_hbm, idx_v, rows_v, sem):
        wid = lax.axis_index("s") * _NC + lax.axis_index("c")
        w0 = wid * b_per_w

        def step(ci, carry):
            base = w0 + ci * ch
            pltpu.sync_copy(idx_hbm.at[pl.ds(base, ch)], idx_v)
            pltpu.async_copy(table_hbm.at[idx_v], rows_v, sem).wait()
            pltpu.sync_copy(rows_v, out_hbm.at[pl.ds(base, ch)])
            return carry

        if n_chunks == 1:
            step(0, 0)
        else:
            lax.fori_loop(0, n_chunks, step, 0)

    return pl.kernel(body,
                     out_type=jax.ShapeDtypeStruct((B, D), jnp.float32),
                     mesh=_mesh(),
                     scratch_types=[pltpu.VMEM((ch,), jnp.int32),
                                    pltpu.VMEM((ch, D), jnp.float32),
                                    pltpu.SemaphoreType.DMA])


def _conv2(G, Wbd, brow, elu, N, bf16):
    D = G.shape[1]
    Do = Wbd.shape[2]
    T = min(2048, N)
    NT = N // T

    def body(g_ref, w_ref, b_ref, o_ref):
        s = pl.program_id(1)
        g = g_ref[...]
        if bf16:
            g = g.astype(jnp.bfloat16)
        part = jnp.dot(g, w_ref[0], preferred_element_type=jnp.float32)

        @pl.when(s == 0)
        def _():
            o_ref[...] = part + b_ref[...]

        @pl.when(s > 0)
        def _():
            o_ref[...] = o_ref[...] + part

        if elu:
            @pl.when(s == _SEQ - 1)
            def _():
                a = o_ref[...]
                o_ref[...] = jnp.where(
                    a > 0, a, jnp.exp(jnp.minimum(a, 0.0)) - 1.0)

    return pl.pallas_call(
        body,
        grid=(NT, _SEQ),
        in_specs=[pl.BlockSpec((T, D), lambda i, s: (s * NT + i, 0)),
                  pl.BlockSpec((1, D, Do), lambda i, s: (s, 0, 0)),
                  pl.BlockSpec((1, Do), lambda i, s: (0, 0))],
        out_specs=pl.BlockSpec((T, Do), lambda i, s: (i, 0)),
        out_shape=jax.ShapeDtypeStruct((N, Do), jnp.float32),
    )(G, Wbd, brow)


def _z2(h, Wbd, N, bf16):
    D = h.shape[1]
    Do = Wbd.shape[2]
    T = min(2048, N)
    NT = N // T

    def body(h_ref, w_ref, o_ref):
        g = h_ref[...]
        if bf16:
            g = g.astype(jnp.bfloat16)
        o_ref[...] = jnp.dot(g, w_ref[0], preferred_element_type=jnp.float32)

    return pl.pallas_call(
        body,
        grid=(NT, _SEQ),
        in_specs=[pl.BlockSpec((T, D), lambda i, s: (i, 0)),
                  pl.BlockSpec((1, D, Do), lambda i, s: (s, 0, 0))],
        out_specs=pl.BlockSpec((T, Do), lambda i, s: (s * NT + i, 0)),
        out_shape=jax.ShapeDtypeStruct((_SEQ * N, Do), jnp.float32),
    )(h, Wbd)


def _sum9b(G, brow, elu, N):
    D = G.shape[1]
    T = min(2048, N)
    NT = N // T

    def body(*refs):
        o_ref = refs[-1]
        b_ref = refs[_SEQ]
        acc = refs[0][...] + b_ref[...]
        for k in range(1, _SEQ):
            acc = acc + refs[k][...]
        if elu:
            acc = jnp.where(acc > 0, acc, jnp.exp(jnp.minimum(acc, 0.0)) - 1.0)
        o_ref[...] = acc

    def mk(k):
        return pl.BlockSpec((T, D), lambda i, k=k: (k * NT + i, 0))

    return pl.pallas_call(
        body,
        grid=(NT,),
        in_specs=[mk(k) for k in range(_SEQ)]
        + [pl.BlockSpec((1, D), lambda i: (0, 0))],
        out_specs=pl.BlockSpec((T, D), lambda i: (i, 0)),
        out_shape=jax.ShapeDtypeStruct((N, D), jnp.float32),
    )(*([G] * _SEQ + [brow]))


def _pool2(G4, val, M):
    D = G4.shape[1]
    T = min(2048, M)
    NT = M // T

    def body(g0, g1, g2, g3, v_ref, o_ref):
        gs = [g0, g1, g2, g3]
        acc = None
        for k in range(4):
            vk = v_ref[k][:, None]
            term = gs[k][...] * vk
            acc = term if acc is None else acc + term
        o_ref[...] = acc

    def mk(k):
        return pl.BlockSpec((T, D), lambda i, k=k: (k * NT + i, 0))

    return pl.pallas_call(
        body,
        grid=(NT,),
        in_specs=[mk(k) for k in range(4)]
        + [pl.BlockSpec((4, T), lambda i: (0, i))],
        out_specs=pl.BlockSpec((T, D), lambda i: (i, 0)),
        out_shape=jax.ShapeDtypeStruct((M, D), jnp.float32),
    )(G4, G4, G4, G4, val)


def _fc_tc(h, w1t, b1, w2t, b2):
    def body(h_ref, w1_ref, b1_ref, w2_ref, b2_ref, o_ref):
        mu = jnp.dot(h_ref[...], w1_ref[...],
                     preferred_element_type=jnp.float32) + b1_ref[...]
        mu = jax.nn.sigmoid(mu)
        o_ref[...] = jnp.dot(mu, w2_ref[...],
                             preferred_element_type=jnp.float32) + b2_ref[...]

    return pl.pallas_call(
        body,
        out_shape=jax.ShapeDtypeStruct((h.shape[0], w2t.shape[1]),
                                       jnp.float32),
    )(h, w1t, b1, w2t, b2)


def _wstack(W, Cin, Cout):
    return W.reshape(Cout, _SEQ, Cin).transpose(1, 2, 0)


def _bd(Ws, bf16):
    eye = jnp.eye(_BS, dtype=jnp.float32)
    out = jax.vmap(lambda w: jnp.kron(eye, w))(Ws)
    return out.astype(jnp.bfloat16) if bf16 else out


def _idx9(sp, N):
    return (jnp.arange(_SEQ, dtype=jnp.int32)[:, None] * N
            + jnp.transpose(sp)).reshape(-1)


def kernel(x, sp_idx_0, sp_idx_1, sp_idx_2, sp_idx_3,
           dn_row_0, dn_row_1, dn_row_2, dn_row_3,
           dn_col_0, dn_col_1, dn_col_2, dn_col_3,
           dn_val_0, dn_val_1, dn_val_2, dn_val_3,
           up_row_0, up_row_1, up_row_2, up_row_3,
           up_col_0, up_col_1, up_col_2, up_col_3,
           up_val_0, up_val_1, up_val_2, up_val_3,
           enW0, enW1, enW2, enW3, enb0, enb1, enb2, enb3,
           en_fcW, en_fcb, de_fcW, de_fcb,
           deW0, deW1, deW2, deW3, deb0, deb1, deb2, deb3,
           outW, outb):
    sp = [sp_idx_0, sp_idx_1, sp_idx_2, sp_idx_3]
    enW = [enW0, enW1, enW2, enW3]
    enb = [enb0, enb1, enb2, enb3]
    deW = [deW0, deW1, deW2, deW3]
    deb = [deb0, deb1, deb2, deb3]
    dncol = [dn_col_0, dn_col_1, dn_col_2, dn_col_3]
    dnval = [dn_val_0, dn_val_1, dn_val_2, dn_val_3]
    upcol = [up_col_0, up_col_1, up_col_2, up_col_3]
    upval = [up_val_0, up_val_1, up_val_2, up_val_3]

    x8 = jnp.pad(jnp.transpose(x, (1, 0, 2)), ((0, 0), (0, 0), (0, 5)))
    h = x8.reshape(_SZ[0], _BS * 8)

    for i in range(4):
        N, Cin, Cout = _SZ[i], _C[i], _C[i + 1]
        bf16 = N >= 4096
        if i == 0:
            Ws = jnp.pad(_wstack(enW[0], Cin, Cout), ((0, 0), (0, 5), (0, 0)))
        else:
            Ws = _wstack(enW[i], Cin, Cout)
        D = h.shape[1]
        G = _sc_gather(D, _SEQ * N)(h, jnp.transpose(sp[i]).reshape(-1))
        conv = _conv2(G, _bd(Ws, bf16),
                      jnp.tile(enb[i], _BS).reshape(1, _BS * Cout),
                      True, N, bf16)
        M = _SZ[i + 1]
        Do = _BS * Cout
        colT = jnp.transpose(dncol[i].reshape(M, 4)).reshape(-1)
        valT = jnp.transpose(dnval[i].reshape(M, 4))
        G4 = _sc_gather(Do, 4 * M)(conv, colT)
        h = _pool2(G4, valT, M)

    hflat = jnp.transpose(h.reshape(_SZ[4], _BS, _C[4]),
                          (1, 0, 2)).reshape(_BS, _SZ[4] * _C[4])
    z = _fc_tc(hflat, en_fcW.T, en_fcb.reshape(1, -1),
               de_fcW.T, de_fcb.reshape(1, -1))
    h = jnp.transpose(z.reshape(_BS, _SZ[4], _C[4]),
                      (1, 0, 2)).reshape(_SZ[4], _BS * _C[4])

    dec_cin = [64, 64, 32, 32]
    dec_cout = [64, 32, 32, 32]
    for j in range(4):
        lvl = 3 - j
        N, M = _SZ[lvl], _SZ[lvl + 1]
        Cin, Cout = dec_cin[j], dec_cout[j]
        D, Do = _BS * Cin, _BS * Cout
        colT = jnp.transpose(upcol[lvl].reshape(N, 4)).reshape(-1)
        valT = jnp.transpose(upval[lvl].reshape(N, 4))
        G4 = _sc_gather(D, 4 * N)(h, colT)
        hp = _pool2(G4, valT, N)
        brow = jnp.tile(deb[j], _BS).reshape(1, Do)
        if Cout < Cin:
            Z = _z2(hp, _bd(_wstack(deW[j], Cin, Cout), True), N, True)
            G = _sc_gather(Do, _SEQ * N)(Z, _idx9(sp[lvl], N))
            h = _sum9b(G, brow, True, N)
        else:
            G = _sc_gather(D, _SEQ * N)(hp, jnp.transpose(sp[lvl])
                                        .reshape(-1))
            h = _conv2(G, _bd(_wstack(deW[j], Cin, Cout), N >= 4096),
                       brow, True, N, N >= 4096)

    N = _SZ[0]
    Wo = jnp.pad(_wstack(outW, 32, 3), ((0, 0), (0, 0), (0, 5)))
    Z = _z2(h, _bd(Wo, True), N, True)
    bias8 = jnp.tile(jnp.pad(outb, (0, 5)), _BS).reshape(1, _BS * 8)
    G = _sc_gather(_BS * 8, _SEQ * N)(Z, _idx9(sp[0], N))
    out = _sum9b(G, bias8, False, N)
    out = out.reshape(N, _BS, 8)[:, :, :3]
    return jnp.transpose(out, (1, 0, 2))


# speedup vs baseline: 4.2758x; 1.0685x over previous
import jax
import jax.numpy as jnp
from jax import lax
from jax.experimental import pallas as pl
from jax.experimental.pallas import tpu as pltpu
from jax.experimental.pallas import tpu_sc as plsc

_NC, _NS = 2, 16
_NW = _NC * _NS
_SZ = [16384, 4096, 1024, 256, 64]
_SEQ = 9
_C = [3, 32, 32, 32, 64]
_BS = 16


def _mesh():
    return plsc.VectorSubcoreMesh(core_axis_name="c", subcore_axis_name="s",
                                  num_cores=_NC, num_subcores=_NS)


def _chunk(b_per_w, D, budget):
    for c in range(min(128, b_per_w), 0, -8):
        if b_per_w % c == 0 and c * D * 4 <= budget:
            return c
    raise AssertionError((b_per_w, D))


def _sc_gather(D, B):
    b_per_w = B // _NW
    assert B % _NW == 0 and b_per_w % 8 == 0, (B,)
    ch = _chunk(b_per_w, D, 196608)
    n_chunks = b_per_w // ch

    def body(table_hbm, idx_hbm, out_hbm, idx_v, r0, r1, sem0, sem1):
        wid = lax.axis_index("s") * _NC + lax.axis_index("c")
        w0 = wid * b_per_w
        pltpu.sync_copy(idx_hbm.at[pl.ds(w0, b_per_w)], idx_v)

        def enq(c, buf, sem):
            pltpu.async_copy(table_hbm.at[idx_v.at[pl.ds(c * ch, ch)]],
                             buf, sem)

        def fin(c, buf, sem):
            pltpu.make_async_copy(
                table_hbm.at[idx_v.at[pl.ds(c * ch, ch)]], buf, sem).wait()
            pltpu.sync_copy(buf, out_hbm.at[pl.ds(w0 + c * ch, ch)])

        if n_chunks == 1:
            enq(0, r0, sem0)
            fin(0, r0, sem0)
            return

        enq(0, r0, sem0)

        def step(c, carry):
            nxt = c + 1

            @pl.when(jnp.logical_and(nxt < n_chunks, nxt % 2 == 1))
            def _():
                enq(nxt, r1, sem1)

            @pl.when(jnp.logical_and(nxt < n_chunks, nxt % 2 == 0))
            def _():
                enq(nxt, r0, sem0)

            @pl.when(c % 2 == 0)
            def _():
                fin(c, r0, sem0)

            @pl.when(c % 2 == 1)
            def _():
                fin(c, r1, sem1)

            return carry

        lax.fori_loop(0, n_chunks, step, 0)

    return pl.kernel(body,
                     out_type=jax.ShapeDtypeStruct((B, D), jnp.float32),
                     mesh=_mesh(),
                     scratch_types=[pltpu.VMEM((b_per_w,), jnp.int32),
                                    pltpu.VMEM((ch, D), jnp.float32),
                                    pltpu.VMEM((ch, D), jnp.float32),
                                    pltpu.SemaphoreType.DMA,
                                    pltpu.SemaphoreType.DMA])


def _conv2(G, Wbd, brow, elu, N, bf16):
    D = G.shape[1]
    Do = Wbd.shape[2]
    T = min(2048, N)
    NT = N // T

    def body(g_ref, w_ref, b_ref, o_ref):
        s = pl.program_id(1)
        g = g_ref[...]
        if bf16:
            g = g.astype(jnp.bfloat16)
        part = jnp.dot(g, w_ref[0], preferred_element_type=jnp.float32)

        @pl.when(s == 0)
        def _():
            o_ref[...] = part + b_ref[...]

        @pl.when(s > 0)
        def _():
            o_ref[...] = o_ref[...] + part

        if elu:
            @pl.when(s == _SEQ - 1)
            def _():
                a = o_ref[...]
                o_ref[...] = jnp.where(
                    a > 0, a, jnp.exp(jnp.minimum(a, 0.0)) - 1.0)

    return pl.pallas_call(
        body,
        grid=(NT, _SEQ),
        in_specs=[pl.BlockSpec((T, D), lambda i, s: (s * NT + i, 0)),
                  pl.BlockSpec((1, D, Do), lambda i, s: (s, 0, 0)),
                  pl.BlockSpec((1, Do), lambda i, s: (0, 0))],
        out_specs=pl.BlockSpec((T, Do), lambda i, s: (i, 0)),
        out_shape=jax.ShapeDtypeStruct((N, Do), jnp.float32),
    )(G, Wbd, brow)


def _z2(h, Wbd, N, bf16):
    D = h.shape[1]
    Do = Wbd.shape[2]
    T = min(2048, N)
    NT = N // T

    def body(h_ref, w_ref, o_ref):
        g = h_ref[...]
        if bf16:
            g = g.astype(jnp.bfloat16)
        o_ref[...] = jnp.dot(g, w_ref[0], preferred_element_type=jnp.float32)

    return pl.pallas_call(
        body,
        grid=(NT, _SEQ),
        in_specs=[pl.BlockSpec((T, D), lambda i, s: (i, 0)),
                  pl.BlockSpec((1, D, Do), lambda i, s: (s, 0, 0))],
        out_specs=pl.BlockSpec((T, Do), lambda i, s: (s * NT + i, 0)),
        out_shape=jax.ShapeDtypeStruct((_SEQ * N, Do), jnp.float32),
    )(h, Wbd)


def _sum9b(G, brow, elu, N):
    D = G.shape[1]
    T = min(2048, N)
    NT = N // T

    def body(*refs):
        o_ref = refs[-1]
        b_ref = refs[_SEQ]
        acc = refs[0][...] + b_ref[...]
        for k in range(1, _SEQ):
            acc = acc + refs[k][...]
        if elu:
            acc = jnp.where(acc > 0, acc, jnp.exp(jnp.minimum(acc, 0.0)) - 1.0)
        o_ref[...] = acc

    def mk(k):
        return pl.BlockSpec((T, D), lambda i, k=k: (k * NT + i, 0))

    return pl.pallas_call(
        body,
        grid=(NT,),
        in_specs=[mk(k) for k in range(_SEQ)]
        + [pl.BlockSpec((1, D), lambda i: (0, 0))],
        out_specs=pl.BlockSpec((T, D), lambda i: (i, 0)),
        out_shape=jax.ShapeDtypeStruct((N, D), jnp.float32),
    )(*([G] * _SEQ + [brow]))


def _pool2(G4, val, M):
    D = G4.shape[1]
    T = min(2048, M)
    NT = M // T

    def body(g0, g1, g2, g3, v_ref, o_ref):
        gs = [g0, g1, g2, g3]
        acc = None
        for k in range(4):
            vk = v_ref[k][:, None]
            term = gs[k][...] * vk
            acc = term if acc is None else acc + term
        o_ref[...] = acc

    def mk(k):
        return pl.BlockSpec((T, D), lambda i, k=k: (k * NT + i, 0))

    return pl.pallas_call(
        body,
        grid=(NT,),
        in_specs=[mk(k) for k in range(4)]
        + [pl.BlockSpec((4, T), lambda i: (0, i))],
        out_specs=pl.BlockSpec((T, D), lambda i: (i, 0)),
        out_shape=jax.ShapeDtypeStruct((M, D), jnp.float32),
    )(G4, G4, G4, G4, val)


def _fc_tc(h, w1t, b1, w2t, b2):
    def body(h_ref, w1_ref, b1_ref, w2_ref, b2_ref, o_ref):
        mu = jnp.dot(h_ref[...], w1_ref[...],
                     preferred_element_type=jnp.float32) + b1_ref[...]
        mu = jax.nn.sigmoid(mu)
        o_ref[...] = jnp.dot(mu, w2_ref[...],
                             preferred_element_type=jnp.float32) + b2_ref[...]

    return pl.pallas_call(
        body,
        out_shape=jax.ShapeDtypeStruct((h.shape[0], w2t.shape[1]),
                                       jnp.float32),
    )(h, w1t, b1, w2t, b2)


def _wstack(W, Cin, Cout):
    return W.reshape(Cout, _SEQ, Cin).transpose(1, 2, 0)


def _bd(Ws, bf16):
    eye = jnp.eye(_BS, dtype=jnp.float32)
    out = jax.vmap(lambda w: jnp.kron(eye, w))(Ws)
    return out.astype(jnp.bfloat16) if bf16 else out


def _idx9(sp, N):
    return (jnp.arange(_SEQ, dtype=jnp.int32)[:, None] * N
            + jnp.transpose(sp)).reshape(-1)


def kernel(x, sp_idx_0, sp_idx_1, sp_idx_2, sp_idx_3,
           dn_row_0, dn_row_1, dn_row_2, dn_row_3,
           dn_col_0, dn_col_1, dn_col_2, dn_col_3,
           dn_val_0, dn_val_1, dn_val_2, dn_val_3,
           up_row_0, up_row_1, up_row_2, up_row_3,
           up_col_0, up_col_1, up_col_2, up_col_3,
           up_val_0, up_val_1, up_val_2, up_val_3,
           enW0, enW1, enW2, enW3, enb0, enb1, enb2, enb3,
           en_fcW, en_fcb, de_fcW, de_fcb,
           deW0, deW1, deW2, deW3, deb0, deb1, deb2, deb3,
           outW, outb):
    sp = [sp_idx_0, sp_idx_1, sp_idx_2, sp_idx_3]
    enW = [enW0, enW1, enW2, enW3]
    enb = [enb0, enb1, enb2, enb3]
    deW = [deW0, deW1, deW2, deW3]
    deb = [deb0, deb1, deb2, deb3]
    dncol = [dn_col_0, dn_col_1, dn_col_2, dn_col_3]
    dnval = [dn_val_0, dn_val_1, dn_val_2, dn_val_3]
    upcol = [up_col_0, up_col_1, up_col_2, up_col_3]
    upval = [up_val_0, up_val_1, up_val_2, up_val_3]

    x8 = jnp.pad(jnp.transpose(x, (1, 0, 2)), ((0, 0), (0, 0), (0, 5)))
    h = x8.reshape(_SZ[0], _BS * 8)

    for i in range(4):
        N, Cin, Cout = _SZ[i], _C[i], _C[i + 1]
        bf16 = N >= 4096
        if i == 0:
            Ws = jnp.pad(_wstack(enW[0], Cin, Cout), ((0, 0), (0, 5), (0, 0)))
        else:
            Ws = _wstack(enW[i], Cin, Cout)
        D = h.shape[1]
        G = _sc_gather(D, _SEQ * N)(h, jnp.transpose(sp[i]).reshape(-1))
        conv = _conv2(G, _bd(Ws, bf16),
                      jnp.tile(enb[i], _BS).reshape(1, _BS * Cout),
                      True, N, bf16)
        M = _SZ[i + 1]
        Do = _BS * Cout
        colT = jnp.transpose(dncol[i].reshape(M, 4)).reshape(-1)
        valT = jnp.transpose(dnval[i].reshape(M, 4))
        G4 = _sc_gather(Do, 4 * M)(conv, colT)
        h = _pool2(G4, valT, M)

    hflat = jnp.transpose(h.reshape(_SZ[4], _BS, _C[4]),
                          (1, 0, 2)).reshape(_BS, _SZ[4] * _C[4])
    z = _fc_tc(hflat, en_fcW.T, en_fcb.reshape(1, -1),
               de_fcW.T, de_fcb.reshape(1, -1))
    h = jnp.transpose(z.reshape(_BS, _SZ[4], _C[4]),
                      (1, 0, 2)).reshape(_SZ[4], _BS * _C[4])

    dec_cin = [64, 64, 32, 32]
    dec_cout = [64, 32, 32, 32]
    for j in range(4):
        lvl = 3 - j
        N, M = _SZ[lvl], _SZ[lvl + 1]
        Cin, Cout = dec_cin[j], dec_cout[j]
        D, Do = _BS * Cin, _BS * Cout
        colT = jnp.transpose(upcol[lvl].reshape(N, 4)).reshape(-1)
        valT = jnp.transpose(upval[lvl].reshape(N, 4))
        G4 = _sc_gather(D, 4 * N)(h, colT)
        hp = _pool2(G4, valT, N)
        brow = jnp.tile(deb[j], _BS).reshape(1, Do)
        if Cout < Cin:
            Z = _z2(hp, _bd(_wstack(deW[j], Cin, Cout), True), N, True)
            G = _sc_gather(Do, _SEQ * N)(Z, _idx9(sp[lvl], N))
            h = _sum9b(G, brow, True, N)
        else:
            G = _sc_gather(D, _SEQ * N)(hp, jnp.transpose(sp[lvl])
                                        .reshape(-1))
            h = _conv2(G, _bd(_wstack(deW[j], Cin, Cout), N >= 4096),
                       brow, True, N, N >= 4096)

    N = _SZ[0]
    Wo = jnp.pad(_wstack(outW, 32, 3), ((0, 0), (0, 0), (0, 5)))
    Z = _z2(h, _bd(Wo, True), N, True)
    bias8 = jnp.tile(jnp.pad(outb, (0, 5)), _BS).reshape(1, _BS * 8)
    G = _sc_gather(_BS * 8, _SEQ * N)(Z, _idx9(sp[0], N))
    out = _sum9b(G, bias8, False, N)
    out = out.reshape(N, _BS, 8)[:, :, :3]
    return jnp.transpose(out, (1, 0, 2))
